# Initial kernel scaffold; baseline (speedup 1.0000x reference)
import jax
import jax.numpy as jnp
from jax.experimental import pallas as pl


def kernel(inputs, Wg, bg, Wb, bb):
    raise NotImplementedError("write your pallas kernel here")


# speedup vs baseline: 1.3636x; 1.3636x over previous
import functools

import jax
import jax.numpy as jnp
from jax import lax
from jax.experimental import pallas as pl
from jax.experimental.pallas import tpu as pltpu
from jax.experimental.pallas import tpu_sc as plsc

B, S, D = 4, 8192, 768
K = 256
SB = 512
NW = 32
RPW = (B * K) // NW


def _copy_logits_body(x_ref, wg_ref, out_ref, lg_ref):
    x = x_ref[0]
    out_ref[0] = x
    lg = jax.lax.dot_general(
        x, wg_ref[...],
        dimension_numbers=(((1,), (0,)), ((), ())),
        preferred_element_type=jnp.float32,
    )
    lg_ref[...] = lg.reshape(1, 1, 1, SB)


def _copy_logits(inputs, Wg):
    return pl.pallas_call(
        _copy_logits_body,
        grid=(B, S // SB),
        in_specs=[
            pl.BlockSpec((1, SB, D), lambda b, j: (b, j, 0)),
            pl.BlockSpec((D, 1), lambda b, j: (0, 0)),
        ],
        out_specs=[
            pl.BlockSpec((1, SB, D), lambda b, j: (b, j, 0)),
            pl.BlockSpec((1, 1, 1, SB), lambda b, j: (b, j, 0, 0)),
        ],
        out_shape=[
            jax.ShapeDtypeStruct((B, S, D), jnp.float32),
            jax.ShapeDtypeStruct((B, S // SB, 1, SB), jnp.float32),
        ],
        compiler_params=pltpu.CompilerParams(
            dimension_semantics=("arbitrary", "arbitrary"),
        ),
    )(inputs, Wg)


def _topk_body(lg_ref, idx_ref):
    iota = lax.broadcasted_iota(jnp.int32, (B, S), 1)
    kiota = lax.broadcasted_iota(jnp.int32, (B, K), 1)

    def body(k, carry):
        x, acc = carry
        m = jnp.max(x, axis=1, keepdims=True)
        cand = jnp.where(x >= m, iota, S)
        am = jnp.min(cand, axis=1, keepdims=True)
        x = jnp.where(iota == am, -jnp.inf, x)
        acc = jnp.where(kiota == k, am, acc)
        return x, acc

    x0 = lg_ref[...]
    acc0 = jnp.zeros((B, K), jnp.int32)
    _, acc = lax.fori_loop(0, K, body, (x0, acc0))
    boff = lax.broadcasted_iota(jnp.int32, (B, K), 0) * S
    idx_ref[...] = acc + boff


def _topk(logits):
    return pl.pallas_call(
        _topk_body,
        out_shape=jax.ShapeDtypeStruct((B, K), jnp.int32),
    )(logits)


_SC_MESH = plsc.VectorSubcoreMesh(core_axis_name="c", subcore_axis_name="s")


@functools.partial(
    pl.kernel,
    mesh=_SC_MESH,
    out_type=jax.ShapeDtypeStruct((B * K, D), jnp.float32),
    scratch_types=[
        pltpu.VMEM((RPW,), jnp.int32),
        pltpu.VMEM((RPW, D), jnp.float32),
        pltpu.SemaphoreType.DMA,
    ],
)
def _sc_gather(table_hbm, idx_hbm, out_hbm, idx_v, rows_v, sem):
    wid = lax.axis_index("s") * 2 + lax.axis_index("c")
    pltpu.sync_copy(idx_hbm.at[wid], idx_v)
    pltpu.async_copy(table_hbm.at[idx_v], rows_v, sem).wait()
    pltpu.sync_copy(rows_v, out_hbm.at[pl.ds(wid * RPW, RPW)])


def _mm_body(r_ref, w_ref, b_ref, o_ref):
    o_ref[...] = (
        jax.lax.dot_general(
            r_ref[...], w_ref[...],
            dimension_numbers=(((1,), (0,)), ((), ())),
            preferred_element_type=jnp.float32,
        )
        + b_ref[...]
    )


def _matmul(rows, Wb, bb2d):
    return pl.pallas_call(
        _mm_body,
        out_shape=jax.ShapeDtypeStruct((B * K, D), jnp.float32),
    )(rows, Wb, bb2d)


@functools.partial(
    pl.kernel,
    mesh=_SC_MESH,
    scratch_types=[
        pltpu.VMEM((RPW,), jnp.int32),
        pltpu.VMEM((RPW, D), jnp.float32),
        pltpu.SemaphoreType.DMA,
    ],
)
def _sc_scatter(idx_hbm, y_hbm, out_hbm, idx_v, rows_v, sem):
    wid = lax.axis_index("s") * 2 + lax.axis_index("c")
    pltpu.sync_copy(idx_hbm.at[wid], idx_v)
    pltpu.sync_copy(y_hbm.at[pl.ds(wid * RPW, RPW)], rows_v)
    pltpu.async_copy(rows_v, out_hbm.at[idx_v], sem).wait()


def kernel(inputs, Wg, bg, Wb, bb):
    del bg
    out0, logits4 = _copy_logits(inputs, Wg)
    gidx = _topk(logits4.reshape(B, S))
    idx2d = gidx.reshape(NW, RPW)
    rows = _sc_gather(inputs.reshape(B * S, D), idx2d)
    y = _matmul(rows, Wb, bb.reshape(1, D))
    out_ref = jax.new_ref(out0.reshape(B * S, D))
    _sc_scatter(idx2d, y, out_ref)
    return out_ref[...].reshape(B, S, D)
